# baseline (speedup 1.0000x reference)
import jax
import jax.numpy as jnp
from jax.experimental import pallas as pl
from jax.experimental.pallas import tpu as pltpu


def _round_up(v, m):
    return ((v + m - 1) // m) * m


def _make_body(has_bias):
    def _body(x_ref, w_ref, adj_ref, *rest):
        if has_bias:
            bias_ref, o_ref, s_ref = rest
        else:
            o_ref, s_ref = rest

        @pl.when(pl.program_id(0) == 0)
        def _():
            s_ref[...] = jnp.dot(
                x_ref[...].astype(jnp.bfloat16),
                w_ref[...].astype(jnp.bfloat16),
                preferred_element_type=jnp.float32,
            ).astype(jnp.bfloat16)

        acc = jnp.dot(
            adj_ref[...].astype(jnp.bfloat16),
            s_ref[...],
            preferred_element_type=jnp.float32,
        )
        if has_bias:
            acc = acc + bias_ref[...]
        o_ref[...] = acc

    return _body


def kernel(x, adj, weight, bias):
    n, f_in = x.shape
    f_out = weight.shape[1]
    has_bias = bias is not None

    n_pad = _round_up(n, 512)
    f_in_pad = _round_up(f_in, 128)
    f_out_pad = _round_up(f_out, 128)
    if (n_pad, f_in_pad) != (n, f_in):
        x = jnp.pad(x, ((0, n_pad - n), (0, f_in_pad - f_in)))
    if (f_in_pad, f_out_pad) != weight.shape:
        weight = jnp.pad(
            weight, ((0, f_in_pad - f_in), (0, f_out_pad - f_out)))
    if adj.shape != (n_pad, n_pad):
        adj = jnp.pad(adj, ((0, n_pad - n), (0, n_pad - n)))

    tm = 256
    in_specs = [
        pl.BlockSpec((n_pad, f_in_pad), lambda i: (0, 0)),
        pl.BlockSpec((f_in_pad, f_out_pad), lambda i: (0, 0)),
        pl.BlockSpec((tm, n_pad), lambda i: (i, 0)),
    ]
    operands = [x, weight, adj]
    if has_bias:
        bias_p = bias.astype(jnp.float32)
        if f_out_pad != f_out:
            bias_p = jnp.pad(bias_p, (0, f_out_pad - f_out))
        in_specs.append(pl.BlockSpec((1, f_out_pad), lambda i: (0, 0)))
        operands.append(bias_p.reshape(1, f_out_pad))

    out = pl.pallas_call(
        _make_body(has_bias),
        out_shape=jax.ShapeDtypeStruct((n_pad, f_out_pad), jnp.float32),
        grid=(n_pad // tm,),
        in_specs=in_specs,
        out_specs=pl.BlockSpec((tm, f_out_pad), lambda i: (i, 0)),
        scratch_shapes=[pltpu.VMEM((n_pad, f_out_pad), jnp.bfloat16)],
        compiler_params=pltpu.CompilerParams(
            dimension_semantics=("arbitrary",),
            vmem_limit_bytes=60 * 1024 * 1024,
        ),
    )(*operands)

    if (n_pad, f_out_pad) != (n, f_out):
        out = out[:n, :f_out]
    return out


# speedup vs baseline: 1.0035x; 1.0035x over previous
import jax
import jax.numpy as jnp
from jax.experimental import pallas as pl
from jax.experimental.pallas import tpu as pltpu


def _round_up(v, m):
    return ((v + m - 1) // m) * m


def _make_body(has_bias):
    def _body(x_ref, w_ref, adj_ref, *rest):
        if has_bias:
            bias_ref, o_ref, s_ref = rest
        else:
            o_ref, s_ref = rest

        @pl.when(pl.program_id(0) == 0)
        def _():
            s_ref[...] = jnp.dot(
                x_ref[...].astype(jnp.bfloat16),
                w_ref[...].astype(jnp.bfloat16),
                preferred_element_type=jnp.float32,
            ).astype(jnp.bfloat16)

        acc = jnp.dot(
            adj_ref[...].astype(jnp.bfloat16),
            s_ref[...],
            preferred_element_type=jnp.float32,
        )
        if has_bias:
            acc = acc + bias_ref[...]
        o_ref[...] = acc

    return _body


def kernel(x, adj, weight, bias):
    n, f_in = x.shape
    f_out = weight.shape[1]
    has_bias = bias is not None

    n_pad = _round_up(n, 512)
    f_in_pad = _round_up(f_in, 128)
    f_out_pad = _round_up(f_out, 128)
    if (n_pad, f_in_pad) != (n, f_in):
        x = jnp.pad(x, ((0, n_pad - n), (0, f_in_pad - f_in)))
    if (f_in_pad, f_out_pad) != weight.shape:
        weight = jnp.pad(
            weight, ((0, f_in_pad - f_in), (0, f_out_pad - f_out)))
    if adj.shape != (n_pad, n_pad):
        adj = jnp.pad(adj, ((0, n_pad - n), (0, n_pad - n)))

    tm = 512
    in_specs = [
        pl.BlockSpec((n_pad, f_in_pad), lambda i: (0, 0)),
        pl.BlockSpec((f_in_pad, f_out_pad), lambda i: (0, 0)),
        pl.BlockSpec((tm, n_pad), lambda i: (i, 0)),
    ]
    operands = [x, weight, adj]
    if has_bias:
        bias_p = bias.astype(jnp.float32)
        if f_out_pad != f_out:
            bias_p = jnp.pad(bias_p, (0, f_out_pad - f_out))
        in_specs.append(pl.BlockSpec((1, f_out_pad), lambda i: (0, 0)))
        operands.append(bias_p.reshape(1, f_out_pad))

    out = pl.pallas_call(
        _make_body(has_bias),
        out_shape=jax.ShapeDtypeStruct((n_pad, f_out_pad), jnp.float32),
        grid=(n_pad // tm,),
        in_specs=in_specs,
        out_specs=pl.BlockSpec((tm, f_out_pad), lambda i: (i, 0)),
        scratch_shapes=[pltpu.VMEM((n_pad, f_out_pad), jnp.bfloat16)],
        compiler_params=pltpu.CompilerParams(
            dimension_semantics=("arbitrary",),
            vmem_limit_bytes=60 * 1024 * 1024,
        ),
    )(*operands)

    if (n_pad, f_out_pad) != (n, f_out):
        out = out[:n, :f_out]
    return out
